# baseline (speedup 1.0000x reference)
import functools

import jax
import jax.numpy as jnp
from jax import lax
from jax.experimental import pallas as pl
from jax.experimental.pallas import tpu as pltpu
from jax.experimental.pallas import tpu_sc as plsc

NPAIR = 262144
NQ = 200000
NQ_PAD = 229376
CODE_SPACE = 1 << 24
NC = 2
NT = 16
DUMPSZ = NPAIR
ZEROSZ = 32768
DUMP_BASE = CODE_SPACE
ZERO_BASE = CODE_SPACE + DUMPSZ
MAPW = ZERO_BASE + ZEROSZ
ZSPAN = CODE_SPACE // NT
SCW = 128
PAIR_ROWS = NPAIR // SCW
Q_ROWS = NQ_PAD // 128
ROWS_PER_TILE = PAIR_ROWS // NT
QROWS_PER_TILE = Q_ROWS // (NC * NT)

NPTS = 500000
BLK = 8192
GRID_F = 62
NPTS_PAD = GRID_F * BLK

ACCEPTANCE_OVERLAP = 0.1
ACCEPTANCE_RADIUS = 0.1
RMSE_THRESHOLD = 0.2


def _codes_body(gtr, gts, ovl, qr, qs, codes_o, qcodes_o):
    code = gtr[...] * 4096 + gts[...]
    masked = ovl[...] > ACCEPTANCE_OVERLAP
    row = lax.broadcasted_iota(jnp.int32, (PAIR_ROWS, SCW), 0)
    col = lax.broadcasted_iota(jnp.int32, (PAIR_ROWS, SCW), 1)
    pos = row * SCW + col
    codes_o[...] = jnp.where(masked, code, DUMP_BASE + pos)
    qrow = lax.broadcasted_iota(jnp.int32, (Q_ROWS, 128), 0)
    qcol = lax.broadcasted_iota(jnp.int32, (Q_ROWS, 128), 1)
    qpos = qrow * 128 + qcol
    qvalid = qpos < NQ
    qcode = qr[...] * 4096 + qs[...]
    zslot = ZERO_BASE + (qpos & (ZEROSZ - 1))
    qcodes_o[...] = jnp.where(qvalid, qcode, zslot)


def _compute_codes(gtr, gts, ovl, qr, qs):
    return pl.pallas_call(
        _codes_body,
        out_shape=[
            jax.ShapeDtypeStruct((PAIR_ROWS, SCW), jnp.int32),
            jax.ShapeDtypeStruct((Q_ROWS, 128), jnp.int32),
        ],
    )(gtr, gts, ovl, qr, qs)


def _sc_scatter_body(codes_hbm, map_hbm, zbuf, cbuf, ones, semz, sems, sem1):
    tid = lax.axis_index("s")

    pltpu.async_copy(
        codes_hbm.at[pl.ds(tid * ROWS_PER_TILE, ROWS_PER_TILE)], cbuf, sems)

    def _zb(i, _):
        zbuf[pl.ds(i * 16, 16)] = jnp.zeros((16,), jnp.float32)
        return 0
    lax.fori_loop(0, 1024, _zb, 0)

    def _ob(i, _):
        ones[pl.ds(i * 16, 16)] = jnp.full((16,), 1.0, jnp.float32)
        return 0
    lax.fori_loop(0, SCW // 16, _ob, 0)

    zbase = tid * ZSPAN
    def _zc(i, _):
        pltpu.async_copy(zbuf, map_hbm.at[pl.ds(zbase + i * 16384, 16384)],
                         semz)
        return 0
    lax.fori_loop(0, 64, _zc, 0)
    zrspan = ZEROSZ // NT
    zrbase = ZERO_BASE + tid * zrspan
    pltpu.async_copy(zbuf.at[pl.ds(0, zrspan)],
                     map_hbm.at[pl.ds(zrbase, zrspan)], semz)

    def _zd(i, _):
        pltpu.make_async_copy(
            zbuf, map_hbm.at[pl.ds(zbase + i * 16384, 16384)], semz).wait()
        return 0
    lax.fori_loop(0, 64, _zd, 0)
    pltpu.make_async_copy(zbuf.at[pl.ds(0, zrspan)],
                          map_hbm.at[pl.ds(zrbase, zrspan)], semz).wait()

    pltpu.make_async_copy(
        codes_hbm.at[pl.ds(tid * ROWS_PER_TILE, ROWS_PER_TILE)],
        cbuf, sems).wait()

    plsc.subcore_barrier()

    def _fire(j, _):
        pltpu.async_copy(ones, map_hbm.at[cbuf.at[j]], sem1)
        return 0
    lax.fori_loop(0, ROWS_PER_TILE, _fire, 0)

    def _drain(j, _):
        pltpu.make_async_copy(ones, map_hbm.at[cbuf.at[j]], sem1).wait()
        return 0
    lax.fori_loop(0, ROWS_PER_TILE, _drain, 0)


@functools.cache
def _sc_scatter():
    mesh = plsc.VectorSubcoreMesh(
        core_axis_name="c", subcore_axis_name="s",
        num_cores=1, num_subcores=NT)
    return pl.kernel(
        _sc_scatter_body,
        out_type=jax.ShapeDtypeStruct((MAPW,), jnp.float32),
        mesh=mesh,
        scratch_types=[
            pltpu.VMEM((16384,), jnp.float32),
            pltpu.VMEM((ROWS_PER_TILE, SCW), jnp.int32),
            pltpu.VMEM((SCW,), jnp.float32),
            pltpu.SemaphoreType.DMA,
            pltpu.SemaphoreType.DMA,
            pltpu.SemaphoreType.DMA,
        ],
    )


def _sc_gather_body(qcodes_hbm, map_hbm, part_hbm, qbuf, gvals, accv, sems,
                    sem2):
    tid = lax.axis_index("s")
    cid = lax.axis_index("c")
    wid = cid * NT + tid

    pltpu.sync_copy(
        qcodes_hbm.at[pl.ds(wid * QROWS_PER_TILE, QROWS_PER_TILE)], qbuf)

    def _gfire(j, _):
        pltpu.async_copy(map_hbm.at[qbuf.at[j]], gvals.at[j], sem2)
        return 0
    lax.fori_loop(0, QROWS_PER_TILE, _gfire, 0)

    def _gdrain(j, _):
        pltpu.make_async_copy(map_hbm.at[qbuf.at[j]], gvals.at[j], sem2).wait()
        return 0
    lax.fori_loop(0, QROWS_PER_TILE, _gdrain, 0)

    def _row(j, acc):
        g = gvals.at[j]
        def _grp(k, acc2):
            return acc2 + g[pl.ds(k * 16, 16)]
        return lax.fori_loop(0, 8, _grp, acc)
    acc = lax.fori_loop(0, QROWS_PER_TILE, _row,
                        jnp.zeros((16,), jnp.float32))
    accv[...] = acc
    pltpu.sync_copy(accv, part_hbm.at[cid, tid])


@functools.cache
def _sc_gather():
    mesh = plsc.VectorSubcoreMesh(
        core_axis_name="c", subcore_axis_name="s",
        num_cores=NC, num_subcores=NT)
    return pl.kernel(
        _sc_gather_body,
        out_type=jax.ShapeDtypeStruct((NC, NT, 16), jnp.float32),
        mesh=mesh,
        scratch_types=[
            pltpu.VMEM((QROWS_PER_TILE, 128), jnp.int32),
            pltpu.VMEM((QROWS_PER_TILE, 128), jnp.float32),
            pltpu.VMEM((16,), jnp.float32),
            pltpu.SemaphoreType.DMA,
            pltpu.SemaphoreType.DMA,
        ],
    )


def _stream_body(ref_r, srcc_r, srcf_r, tf_r, rl_r, out_r):
    i = pl.program_id(0)

    @pl.when(i == 0)
    def _():
        out_r[...] = jnp.zeros_like(out_r)

    tf = tf_r[...]
    rl = rl_r[...]

    gidx = i * BLK + lax.broadcasted_iota(jnp.int32, (1, BLK), 1)
    valid = gidx < NPTS

    sx = srcc_r[0:1, :]
    sy = srcc_r[1:2, :]
    sz = srcc_r[2:3, :]
    dx = ref_r[0:1, :] - (tf[0, 0] * sx + tf[0, 1] * sy + tf[0, 2] * sz + tf[0, 3])
    dy = ref_r[1:2, :] - (tf[1, 0] * sx + tf[1, 1] * sy + tf[1, 2] * sz + tf[1, 3])
    dz = ref_r[2:3, :] - (tf[2, 0] * sx + tf[2, 1] * sy + tf[2, 2] * sz + tf[2, 3])
    d2 = dx * dx + dy * dy + dz * dz
    nclose = jnp.sum(jnp.where(
        valid & (d2 < ACCEPTANCE_RADIUS * ACCEPTANCE_RADIUS), 1.0, 0.0))

    fx = srcf_r[0:1, :]
    fy = srcf_r[1:2, :]
    fz = srcf_r[2:3, :]
    ex = rl[0, 0] * fx + rl[0, 1] * fy + rl[0, 2] * fz + rl[0, 3] - fx
    ey = rl[1, 0] * fx + rl[1, 1] * fy + rl[1, 2] * fz + rl[1, 3] - fy
    ez = rl[2, 0] * fx + rl[2, 1] * fy + rl[2, 2] * fz + rl[2, 3] - fz
    rn = jnp.sqrt(ex * ex + ey * ey + ez * ez)
    rsum = jnp.sum(jnp.where(valid, rn, 0.0))

    lanes = lax.broadcasted_iota(jnp.int32, (1, 128), 1)
    out_r[...] += (jnp.where(lanes == 0, nclose, 0.0)
                   + jnp.where(lanes == 1, rsum, 0.0))


def _stream_call(ref_t, srcc_t, srcf_t, tf, rl):
    big = pl.BlockSpec((3, BLK), lambda i: (0, i))
    small4 = pl.BlockSpec((4, 4), lambda i: (0, 0))
    return pl.pallas_call(
        _stream_body,
        grid=(GRID_F,),
        in_specs=[big, big, big, small4, small4],
        out_specs=pl.BlockSpec((1, 128), lambda i: (0, 0)),
        out_shape=jax.ShapeDtypeStruct((1, 128), jnp.float32),
    )(ref_t, srcc_t, srcf_t, tf, rl)


def _final_body(sums_r, part_r, tf_r, est_r, out_r):
    a = sums_r[...]
    lanes = lax.broadcasted_iota(jnp.int32, (1, 128), 1)
    total_close = jnp.sum(jnp.where(lanes == 0, a, 0.0))
    total_rsum = jnp.sum(jnp.where(lanes == 1, a, 0.0))
    f_prec = total_close / NPTS
    rmse = total_rsum / NPTS
    recall = jnp.where(rmse < RMSE_THRESHOLD, 1.0, 0.0)
    c_prec = jnp.sum(part_r[...]) / NQ
    tf = tf_r[...]
    est = est_r[...]
    rte = jnp.sqrt((tf[0, 3] - est[0, 3]) ** 2
                   + (tf[1, 3] - est[1, 3]) ** 2
                   + (tf[2, 3] - est[2, 3]) ** 2)
    tr = (tf[0, 0] * est[0, 0] + tf[0, 1] * est[0, 1] + tf[0, 2] * est[0, 2]
          + tf[1, 0] * est[1, 0] + tf[1, 1] * est[1, 1] + tf[1, 2] * est[1, 2]
          + tf[2, 0] * est[2, 0] + tf[2, 1] * est[2, 1] + tf[2, 2] * est[2, 2])
    x = jnp.clip(0.5 * (tr - 1.0), -1.0, 1.0)
    out_r[...] = (jnp.where(lanes == 0, c_prec, 0.0)
                  + jnp.where(lanes == 1, f_prec, 0.0)
                  + jnp.where(lanes == 2, x, 0.0)
                  + jnp.where(lanes == 3, rte, 0.0)
                  + jnp.where(lanes == 4, rmse, 0.0)
                  + jnp.where(lanes == 5, recall, 0.0))


def _final_call(sums, partials, tf, est):
    return pl.pallas_call(
        _final_body,
        out_shape=jax.ShapeDtypeStruct((1, 128), jnp.float32),
    )(sums, partials, tf, est)


def kernel(ref_points_c, src_points_c, gt_node_corr_overlaps,
           gt_node_corr_indices, ref_node_corr_indices, src_node_corr_indices,
           ref_corr_points, src_corr_points, src_points_f, src_lengths_f,
           transform, estimated_transform):
    gtr = gt_node_corr_indices[:, 0].reshape(PAIR_ROWS, SCW)
    gts = gt_node_corr_indices[:, 1].reshape(PAIR_ROWS, SCW)
    ovl = gt_node_corr_overlaps.reshape(PAIR_ROWS, SCW)
    pad_q = NQ_PAD - NQ
    qr = jnp.pad(ref_node_corr_indices, (0, pad_q)).reshape(Q_ROWS, 128)
    qs = jnp.pad(src_node_corr_indices, (0, pad_q)).reshape(Q_ROWS, 128)

    pad_p = NPTS_PAD - NPTS
    ref_t = jnp.pad(ref_corr_points.T, ((0, 0), (0, pad_p)))
    srcc_t = jnp.pad(src_corr_points.T, ((0, 0), (0, pad_p)))
    srcf_t = jnp.pad(src_points_f.T, ((0, 0), (0, pad_p)))

    tf = transform[0]
    est = estimated_transform[0]
    rl = jnp.linalg.inv(tf) @ est

    codes, qcodes = _compute_codes(gtr, gts, ovl, qr, qs)
    map_hbm = _sc_scatter()(codes)
    partials = _sc_gather()(qcodes, map_hbm)

    sums = _stream_call(ref_t, srcc_t, srcf_t, tf, rl)

    out = _final_call(sums, partials, tf, est)

    rre = jnp.degrees(jnp.arccos(out[0, 2]))
    return jnp.stack([out[0, 0], out[0, 1], rre, out[0, 3], out[0, 4],
                      out[0, 5]])
